# baseline (device time: 215521 ns/iter reference)
import jax
import jax.numpy as jnp
from jax import lax
from jax.experimental import pallas as pl
from jax.experimental.pallas import tpu as pltpu

N_DEV = 16


def kernel(x, w_mat, scale_x, scale_w):
    m_per, k = x.shape
    _, n_per = w_mat.shape
    m_glob = N_DEV * m_per

    def body(x_ref, w_ref, sx_ref, sw_ref, out_ref, gat_ref,
             send_sems, recv_sems):
        my = lax.axis_index("i")
        left = lax.rem(my + (N_DEV - 1), N_DEV)
        right = lax.rem(my + 1, N_DEV)

        barrier_sem = pltpu.get_barrier_semaphore()
        pl.semaphore_signal(barrier_sem, inc=1, device_id=(left,),
                            device_id_type=pl.DeviceIdType.MESH)
        pl.semaphore_signal(barrier_sem, inc=1, device_id=(right,),
                            device_id_type=pl.DeviceIdType.MESH)
        pl.semaphore_wait(barrier_sem, 2)

        scale = sx_ref[0] * sw_ref[0]

        def gemm_store(chunk, row0):
            acc = lax.dot_general(
                chunk, w_ref[:, :], (((1,), (0,)), ((), ())),
                preferred_element_type=jnp.int32)
            y = jnp.maximum(acc.astype(jnp.float32) * scale, 0.0)
            out_ref[pl.ds(row0, m_per), :] = y

        gemm_store(x_ref[:, :], my * m_per)

        for h in range(N_DEV - 1):
            src_origin = lax.rem(my - h + N_DEV, N_DEV)
            src = x_ref if h == 0 else gat_ref.at[
                pl.ds(src_origin * m_per, m_per), :]
            rdma = pltpu.make_async_remote_copy(
                src_ref=src,
                dst_ref=gat_ref.at[pl.ds(src_origin * m_per, m_per), :],
                send_sem=send_sems.at[h],
                recv_sem=recv_sems.at[h],
                device_id=(right,),
                device_id_type=pl.DeviceIdType.MESH,
            )
            rdma.start()
            rdma.wait()
            recv_origin = lax.rem(my - h - 1 + N_DEV, N_DEV)
            chunk = gat_ref[pl.ds(recv_origin * m_per, m_per), :]
            gemm_store(chunk, recv_origin * m_per)

    return pl.pallas_call(
        body,
        out_shape=jax.ShapeDtypeStruct((m_glob, n_per), jnp.float32),
        in_specs=[
            pl.BlockSpec(memory_space=pltpu.VMEM),
            pl.BlockSpec(memory_space=pltpu.VMEM),
            pl.BlockSpec(memory_space=pltpu.SMEM),
            pl.BlockSpec(memory_space=pltpu.SMEM),
        ],
        out_specs=pl.BlockSpec(memory_space=pltpu.VMEM),
        scratch_shapes=[
            pltpu.VMEM((m_glob, k), jnp.int8),
            pltpu.SemaphoreType.DMA((N_DEV - 1,)),
            pltpu.SemaphoreType.DMA((N_DEV - 1,)),
        ],
        compiler_params=pltpu.CompilerParams(collective_id=0),
    )(x, w_mat, scale_x, scale_w)


# device time: 111334 ns/iter; 1.9358x vs baseline; 1.9358x over previous
import jax
import jax.numpy as jnp
from jax import lax
from jax.experimental import pallas as pl
from jax.experimental.pallas import tpu as pltpu

N_DEV = 16
CW_HOPS = 8
CCW_HOPS = 7


def _ring_to_logical(r):
    c = r // 4
    m = lax.rem(r, 4)
    z = jnp.where(lax.rem(c, 2) == 0, m, 3 - m)
    return z * 4 + c


def _logical_to_ring(l):
    c = lax.rem(l, 4)
    z = l // 4
    m = jnp.where(lax.rem(c, 2) == 0, z, 3 - z)
    return c * 4 + m


def kernel(x, w_mat, scale_x, scale_w):
    m_per, k = x.shape
    _, n_per = w_mat.shape
    m_glob = N_DEV * m_per

    def body(x_ref, w_ref, sx_ref, sw_ref, out_ref, gat_ref,
             cw_send_sems, cw_recv_sems, ccw_send_sems, ccw_recv_sems):
        my = lax.axis_index("i")
        rpos = _logical_to_ring(my)
        left = _ring_to_logical(lax.rem(rpos + (N_DEV - 1), N_DEV))
        right = _ring_to_logical(lax.rem(rpos + 1, N_DEV))

        barrier_sem = pltpu.get_barrier_semaphore()
        pl.semaphore_signal(barrier_sem, inc=1, device_id=(left,),
                            device_id_type=pl.DeviceIdType.MESH)
        pl.semaphore_signal(barrier_sem, inc=1, device_id=(right,),
                            device_id_type=pl.DeviceIdType.MESH)
        pl.semaphore_wait(barrier_sem, 2)

        scale = sx_ref[0] * sw_ref[0]

        def slot(ring_origin):
            return pl.ds(ring_origin * m_per, m_per)

        def send_cw(h):
            q = lax.rem(rpos - h + N_DEV, N_DEV)
            src = x_ref if h == 0 else gat_ref.at[slot(q), :]
            r = pltpu.make_async_remote_copy(
                src_ref=src,
                dst_ref=gat_ref.at[slot(q), :],
                send_sem=cw_send_sems.at[h],
                recv_sem=cw_recv_sems.at[h],
                device_id=(right,),
                device_id_type=pl.DeviceIdType.MESH,
            )
            r.start()
            return r

        def send_ccw(h):
            q = lax.rem(rpos + h, N_DEV)
            src = x_ref if h == 0 else gat_ref.at[slot(q), :]
            r = pltpu.make_async_remote_copy(
                src_ref=src,
                dst_ref=gat_ref.at[slot(q), :],
                send_sem=ccw_send_sems.at[h],
                recv_sem=ccw_recv_sems.at[h],
                device_id=(left,),
                device_id_type=pl.DeviceIdType.MESH,
            )
            r.start()
            return r

        def gemm_store(chunk, logical_origin):
            acc = lax.dot_general(
                chunk, w_ref[:, :], (((1,), (0,)), ((), ())),
                preferred_element_type=jnp.int32)
            y = jnp.maximum(acc.astype(jnp.float32) * scale, 0.0)
            out_ref[pl.ds(logical_origin * m_per, m_per), :] = y

        cw = [send_cw(0)]
        ccw = [send_ccw(0)]

        gemm_store(x_ref[:, :], my)

        for h in range(CW_HOPS):
            cw_q = lax.rem(rpos - h - 1 + N_DEV, N_DEV)
            cw[h].wait_recv()
            if h + 1 < CW_HOPS:
                cw.append(send_cw(h + 1))

            if h < CCW_HOPS:
                ccw_q = lax.rem(rpos + h + 1, N_DEV)
                ccw[h].wait_recv()
                if h + 1 < CCW_HOPS:
                    ccw.append(send_ccw(h + 1))

            gemm_store(gat_ref[slot(cw_q), :], _ring_to_logical(cw_q))
            if h < CCW_HOPS:
                gemm_store(gat_ref[slot(ccw_q), :], _ring_to_logical(ccw_q))

        for r in cw + ccw:
            r.wait_send()

    return pl.pallas_call(
        body,
        out_shape=jax.ShapeDtypeStruct((m_glob, n_per), jnp.float32),
        in_specs=[
            pl.BlockSpec(memory_space=pltpu.VMEM),
            pl.BlockSpec(memory_space=pltpu.VMEM),
            pl.BlockSpec(memory_space=pltpu.SMEM),
            pl.BlockSpec(memory_space=pltpu.SMEM),
        ],
        out_specs=pl.BlockSpec(memory_space=pltpu.VMEM),
        scratch_shapes=[
            pltpu.VMEM((m_glob, k), jnp.int8),
            pltpu.SemaphoreType.DMA((CW_HOPS,)),
            pltpu.SemaphoreType.DMA((CW_HOPS,)),
            pltpu.SemaphoreType.DMA((CCW_HOPS,)),
            pltpu.SemaphoreType.DMA((CCW_HOPS,)),
        ],
        compiler_params=pltpu.CompilerParams(collective_id=0),
    )(x, w_mat, scale_x, scale_w)


# device time: 100560 ns/iter; 2.1432x vs baseline; 1.1071x over previous
import jax
import jax.numpy as jnp
from jax import lax
from jax.experimental import pallas as pl
from jax.experimental.pallas import tpu as pltpu

N_DEV = 16
CW_HOPS = 8
CCW_HOPS = 7
NSUB = 2


def _ring_to_logical(r):
    c = r // 4
    m = lax.rem(r, 4)
    z = jnp.where(lax.rem(c, 2) == 0, m, 3 - m)
    return z * 4 + c


def _logical_to_ring(l):
    c = lax.rem(l, 4)
    z = l // 4
    m = jnp.where(lax.rem(c, 2) == 0, z, 3 - z)
    return c * 4 + m


def kernel(x, w_mat, scale_x, scale_w):
    m_per, k = x.shape
    _, n_per = w_mat.shape
    m_glob = N_DEV * m_per

    def body(x_ref, w_ref, sx_ref, sw_ref, out_ref, gat_ref,
             cw_send_sems, cw_recv_sems, ccw_send_sem, ccw_recv_sems):
        my = lax.axis_index("i")
        rpos = _logical_to_ring(my)
        left = _ring_to_logical(lax.rem(rpos + (N_DEV - 1), N_DEV))
        right = _ring_to_logical(lax.rem(rpos + 1, N_DEV))

        barrier_sem = pltpu.get_barrier_semaphore()
        pl.semaphore_signal(barrier_sem, inc=1, device_id=(left,),
                            device_id_type=pl.DeviceIdType.MESH)
        pl.semaphore_signal(barrier_sem, inc=1, device_id=(right,),
                            device_id_type=pl.DeviceIdType.MESH)
        pl.semaphore_wait(barrier_sem, 2)

        scale = sx_ref[0] * sw_ref[0]
        hs = m_per // NSUB

        def sub(ring_origin, j):
            return pl.ds(ring_origin * m_per + j * hs, hs)

        def send_cw(h, j):
            q = lax.rem(rpos - h + N_DEV, N_DEV)
            src = (x_ref.at[pl.ds(j * hs, hs), :] if h == 0
                   else gat_ref.at[sub(q, j), :])
            r = pltpu.make_async_remote_copy(
                src_ref=src,
                dst_ref=gat_ref.at[sub(q, j), :],
                send_sem=cw_send_sems.at[j],
                recv_sem=cw_recv_sems.at[h * NSUB + j],
                device_id=(right,),
                device_id_type=pl.DeviceIdType.MESH,
            )
            r.start()
            return r

        def send_ccw(h):
            q = lax.rem(rpos + h, N_DEV)
            src = x_ref if h == 0 else gat_ref.at[pl.ds(q * m_per, m_per), :]
            r = pltpu.make_async_remote_copy(
                src_ref=src,
                dst_ref=gat_ref.at[pl.ds(q * m_per, m_per), :],
                send_sem=ccw_send_sem.at[0],
                recv_sem=ccw_recv_sems.at[h],
                device_id=(left,),
                device_id_type=pl.DeviceIdType.MESH,
            )
            r.start()
            return r

        def gemm_store(chunk, logical_origin):
            acc = lax.dot_general(
                chunk, w_ref[:, :], (((1,), (0,)), ((), ())),
                preferred_element_type=jnp.int32)
            y = jnp.maximum(acc.astype(jnp.float32) * scale, 0.0)
            out_ref[pl.ds(logical_origin * m_per, m_per), :] = y

        cw = {(0, j): send_cw(0, j) for j in range(NSUB)}
        ccw = [send_ccw(0)]

        gemm_store(x_ref[:, :], my)

        for h in range(CW_HOPS):
            cw_q = lax.rem(rpos - h - 1 + N_DEV, N_DEV)
            for j in range(NSUB):
                cw[(h, j)].wait_recv()
                if h + 1 < CW_HOPS:
                    cw[(h, j)].wait_send()
                    cw[(h + 1, j)] = send_cw(h + 1, j)

            if h < CCW_HOPS:
                ccw_q = lax.rem(rpos + h + 1, N_DEV)
                ccw[h].wait_recv()
                if h + 1 < CCW_HOPS:
                    ccw[h].wait_send()
                    ccw.append(send_ccw(h + 1))

            gemm_store(gat_ref[pl.ds(cw_q * m_per, m_per), :],
                       _ring_to_logical(cw_q))
            if h < CCW_HOPS:
                gemm_store(gat_ref[pl.ds(ccw_q * m_per, m_per), :],
                           _ring_to_logical(ccw_q))

        for j in range(NSUB):
            cw[(CW_HOPS - 1, j)].wait_send()
        ccw[CCW_HOPS - 1].wait_send()

    return pl.pallas_call(
        body,
        out_shape=jax.ShapeDtypeStruct((m_glob, n_per), jnp.float32),
        in_specs=[
            pl.BlockSpec(memory_space=pltpu.VMEM),
            pl.BlockSpec(memory_space=pltpu.VMEM),
            pl.BlockSpec(memory_space=pltpu.SMEM),
            pl.BlockSpec(memory_space=pltpu.SMEM),
        ],
        out_specs=pl.BlockSpec(memory_space=pltpu.VMEM),
        scratch_shapes=[
            pltpu.VMEM((m_glob, k), jnp.int8),
            pltpu.SemaphoreType.DMA((NSUB,)),
            pltpu.SemaphoreType.DMA((CW_HOPS * NSUB,)),
            pltpu.SemaphoreType.DMA((1,)),
            pltpu.SemaphoreType.DMA((CCW_HOPS,)),
        ],
        compiler_params=pltpu.CompilerParams(collective_id=0),
    )(x, w_mat, scale_x, scale_w)


# device time: 99275 ns/iter; 2.1709x vs baseline; 1.0129x over previous
import jax
import jax.numpy as jnp
from jax import lax
from jax.experimental import pallas as pl
from jax.experimental.pallas import tpu as pltpu

N_DEV = 16
CW_HOPS = 8
CCW_HOPS = 7
NSUB = 2


def _ring_to_logical(r):
    c = r // 4
    m = lax.rem(r, 4)
    z = jnp.where(lax.rem(c, 2) == 0, m, 3 - m)
    return z * 4 + c


def _logical_to_ring(l):
    c = lax.rem(l, 4)
    z = l // 4
    m = jnp.where(lax.rem(c, 2) == 0, z, 3 - z)
    return c * 4 + m


def kernel(x, w_mat, scale_x, scale_w):
    m_per, k = x.shape
    _, n_per = w_mat.shape
    m_glob = N_DEV * m_per

    def body(x_ref, w_ref, sx_ref, sw_ref, out_ref, gat_ref,
             cw_send_sems, cw_recv_sems, ccw_send_sems, ccw_recv_sems):
        my = lax.axis_index("i")
        rpos = _logical_to_ring(my)
        left = _ring_to_logical(lax.rem(rpos + (N_DEV - 1), N_DEV))
        right = _ring_to_logical(lax.rem(rpos + 1, N_DEV))

        barrier_sem = pltpu.get_barrier_semaphore()
        pl.semaphore_signal(barrier_sem, inc=1, device_id=(left,),
                            device_id_type=pl.DeviceIdType.MESH)
        pl.semaphore_signal(barrier_sem, inc=1, device_id=(right,),
                            device_id_type=pl.DeviceIdType.MESH)
        pl.semaphore_wait(barrier_sem, 2)

        scale = sx_ref[0] * sw_ref[0]
        hs = m_per // NSUB

        def sub(ring_origin, j):
            return pl.ds(ring_origin * m_per + j * hs, hs)

        def send_cw(h, j):
            q = lax.rem(rpos - h + N_DEV, N_DEV)
            src = (x_ref.at[pl.ds(j * hs, hs), :] if h == 0
                   else gat_ref.at[sub(q, j), :])
            r = pltpu.make_async_remote_copy(
                src_ref=src,
                dst_ref=gat_ref.at[sub(q, j), :],
                send_sem=cw_send_sems.at[j],
                recv_sem=cw_recv_sems.at[h * NSUB + j],
                device_id=(right,),
                device_id_type=pl.DeviceIdType.MESH,
            )
            r.start()
            return r

        def send_ccw(h, j):
            q = lax.rem(rpos + h, N_DEV)
            src = (x_ref.at[pl.ds(j * hs, hs), :] if h == 0
                   else gat_ref.at[sub(q, j), :])
            r = pltpu.make_async_remote_copy(
                src_ref=src,
                dst_ref=gat_ref.at[sub(q, j), :],
                send_sem=ccw_send_sems.at[j],
                recv_sem=ccw_recv_sems.at[h * NSUB + j],
                device_id=(left,),
                device_id_type=pl.DeviceIdType.MESH,
            )
            r.start()
            return r

        def gemm_store(chunk, logical_origin):
            acc = lax.dot_general(
                chunk, w_ref[:, :], (((1,), (0,)), ((), ())),
                preferred_element_type=jnp.int32)
            y = jnp.maximum(acc.astype(jnp.float32) * scale, 0.0)
            out_ref[pl.ds(logical_origin * m_per, m_per), :] = y

        cw = {(0, j): send_cw(0, j) for j in range(NSUB)}
        ccw = {(0, j): send_ccw(0, j) for j in range(NSUB)}

        gemm_store(x_ref[:, :], my)

        for h in range(CW_HOPS):
            cw_q = lax.rem(rpos - h - 1 + N_DEV, N_DEV)
            for j in range(NSUB):
                cw[(h, j)].wait_recv()
                if h + 1 < CW_HOPS:
                    cw[(h, j)].wait_send()
                    cw[(h + 1, j)] = send_cw(h + 1, j)

            if h < CCW_HOPS:
                ccw_q = lax.rem(rpos + h + 1, N_DEV)
                for j in range(NSUB):
                    ccw[(h, j)].wait_recv()
                    if h + 1 < CCW_HOPS:
                        ccw[(h, j)].wait_send()
                        ccw[(h + 1, j)] = send_ccw(h + 1, j)

            gemm_store(gat_ref[pl.ds(cw_q * m_per, m_per), :],
                       _ring_to_logical(cw_q))
            if h < CCW_HOPS:
                gemm_store(gat_ref[pl.ds(ccw_q * m_per, m_per), :],
                           _ring_to_logical(ccw_q))

        for j in range(NSUB):
            cw[(CW_HOPS - 1, j)].wait_send()
            ccw[(CCW_HOPS - 1, j)].wait_send()

    return pl.pallas_call(
        body,
        out_shape=jax.ShapeDtypeStruct((m_glob, n_per), jnp.float32),
        in_specs=[
            pl.BlockSpec(memory_space=pltpu.VMEM),
            pl.BlockSpec(memory_space=pltpu.VMEM),
            pl.BlockSpec(memory_space=pltpu.SMEM),
            pl.BlockSpec(memory_space=pltpu.SMEM),
        ],
        out_specs=pl.BlockSpec(memory_space=pltpu.VMEM),
        scratch_shapes=[
            pltpu.VMEM((m_glob, k), jnp.int8),
            pltpu.SemaphoreType.DMA((NSUB,)),
            pltpu.SemaphoreType.DMA((CW_HOPS * NSUB,)),
            pltpu.SemaphoreType.DMA((NSUB,)),
            pltpu.SemaphoreType.DMA((CCW_HOPS * NSUB,)),
        ],
        compiler_params=pltpu.CompilerParams(collective_id=0),
    )(x, w_mat, scale_x, scale_w)


# device time: 97048 ns/iter; 2.2208x vs baseline; 1.0229x over previous
import jax
import jax.numpy as jnp
from jax import lax
from jax.experimental import pallas as pl
from jax.experimental.pallas import tpu as pltpu

N_DEV = 16
HOPS = 8
NSUB = 2


def _ring_to_logical(r):
    c = r // 4
    m = lax.rem(r, 4)
    z = jnp.where(lax.rem(c, 2) == 0, m, 3 - m)
    return z * 4 + c


def _logical_to_ring(l):
    c = lax.rem(l, 4)
    z = l // 4
    m = jnp.where(lax.rem(c, 2) == 0, z, 3 - z)
    return c * 4 + m


def kernel(x, w_mat, scale_x, scale_w):
    m_per, k = x.shape
    _, n_per = w_mat.shape
    m_glob = N_DEV * m_per

    def body(x_ref, w_ref, sx_ref, sw_ref, out_ref, gat_ref,
             cw_send_sems, cw_recv_sems, ccw_send_sems, ccw_recv_sems):
        my = lax.axis_index("i")
        rpos = _logical_to_ring(my)
        left = _ring_to_logical(lax.rem(rpos + (N_DEV - 1), N_DEV))
        right = _ring_to_logical(lax.rem(rpos + 1, N_DEV))

        barrier_sem = pltpu.get_barrier_semaphore()
        pl.semaphore_signal(barrier_sem, inc=1, device_id=(left,),
                            device_id_type=pl.DeviceIdType.MESH)
        pl.semaphore_signal(barrier_sem, inc=1, device_id=(right,),
                            device_id_type=pl.DeviceIdType.MESH)
        pl.semaphore_wait(barrier_sem, 2)

        scale = sx_ref[0] * sw_ref[0]
        hs = m_per // NSUB

        def sub(ring_origin, j):
            return pl.ds(ring_origin * m_per + j * hs, hs)

        def send_cw(h, j):
            q = lax.rem(rpos - h + N_DEV, N_DEV)
            src = (x_ref.at[pl.ds(j * hs, hs), :] if h == 0
                   else gat_ref.at[sub(q, j), :])
            r = pltpu.make_async_remote_copy(
                src_ref=src,
                dst_ref=gat_ref.at[sub(q, j), :],
                send_sem=cw_send_sems.at[j],
                recv_sem=cw_recv_sems.at[h * NSUB + j],
                device_id=(right,),
                device_id_type=pl.DeviceIdType.MESH,
            )
            r.start()
            return r

        def send_ccw(h, j):
            q = lax.rem(rpos + h, N_DEV)
            src = (x_ref.at[pl.ds(j * hs, hs), :] if h == 0
                   else gat_ref.at[sub(q, j), :])
            idx = h * NSUB + j if h < 7 else 14
            r = pltpu.make_async_remote_copy(
                src_ref=src,
                dst_ref=gat_ref.at[sub(q, j), :],
                send_sem=ccw_send_sems.at[j],
                recv_sem=ccw_recv_sems.at[idx],
                device_id=(left,),
                device_id_type=pl.DeviceIdType.MESH,
            )
            r.start()
            return r

        def gemm_store(chunk, logical_origin):
            acc = lax.dot_general(
                chunk, w_ref[:, :], (((1,), (0,)), ((), ())),
                preferred_element_type=jnp.int32)
            y = jnp.maximum(acc.astype(jnp.float32) * scale, 0.0)
            out_ref[pl.ds(logical_origin * m_per, m_per), :] = y

        cw = {(0, j): send_cw(0, j) for j in range(NSUB)}
        ccw = {(0, j): send_ccw(0, j) for j in range(NSUB)}

        gemm_store(x_ref[:, :], my)

        def hop_halves(h, last_j):
            return [0, 1] if h < 7 else [last_j]

        for h in range(HOPS):
            for j in hop_halves(h, 0):
                cw[(h, j)].wait_recv()
                if h + 1 < HOPS and j in hop_halves(h + 1, 0):
                    cw[(h, j)].wait_send()
                    cw[(h + 1, j)] = send_cw(h + 1, j)

            for j in hop_halves(h, 1):
                ccw[(h, j)].wait_recv()
                if h + 1 < HOPS and j in hop_halves(h + 1, 1):
                    ccw[(h, j)].wait_send()
                    ccw[(h + 1, j)] = send_ccw(h + 1, j)

            if h < 7:
                cw_q = lax.rem(rpos - h - 1 + N_DEV, N_DEV)
                ccw_q = lax.rem(rpos + h + 1, N_DEV)
                gemm_store(gat_ref[pl.ds(cw_q * m_per, m_per), :],
                           _ring_to_logical(cw_q))
                gemm_store(gat_ref[pl.ds(ccw_q * m_per, m_per), :],
                           _ring_to_logical(ccw_q))
            else:
                q8 = lax.rem(rpos + 8, N_DEV)
                gemm_store(gat_ref[pl.ds(q8 * m_per, m_per), :],
                           _ring_to_logical(q8))

        cw[(7, 0)].wait_send()
        cw[(6, 1)].wait_send()
        ccw[(7, 1)].wait_send()
        ccw[(6, 0)].wait_send()

    return pl.pallas_call(
        body,
        out_shape=jax.ShapeDtypeStruct((m_glob, n_per), jnp.float32),
        in_specs=[
            pl.BlockSpec(memory_space=pltpu.VMEM),
            pl.BlockSpec(memory_space=pltpu.VMEM),
            pl.BlockSpec(memory_space=pltpu.SMEM),
            pl.BlockSpec(memory_space=pltpu.SMEM),
        ],
        out_specs=pl.BlockSpec(memory_space=pltpu.VMEM),
        scratch_shapes=[
            pltpu.VMEM((m_glob, k), jnp.int8),
            pltpu.SemaphoreType.DMA((NSUB,)),
            pltpu.SemaphoreType.DMA((15,)),
            pltpu.SemaphoreType.DMA((NSUB,)),
            pltpu.SemaphoreType.DMA((15,)),
        ],
        compiler_params=pltpu.CompilerParams(collective_id=0),
    )(x, w_mat, scale_x, scale_w)
